# Initial kernel scaffold; baseline (speedup 1.0000x reference)
import jax
import jax.numpy as jnp
from jax.experimental import pallas as pl


def kernel(x, W_fcsn, b_fcsn, W_score, b_score, W1, b1):
    raise NotImplementedError("write your pallas kernel here")


# speedup vs baseline: 1.1018x; 1.1018x over previous
import functools

import jax
import jax.numpy as jnp
from jax import lax
from jax.experimental import pallas as pl
from jax.experimental.pallas import tpu as pltpu
from jax.experimental.pallas import tpu_sc as plsc

C = 256
T = 8192
TILE = 512
NTILE = T // TILE
TPAD = (NTILE + 1) * TILE
K = int(T * 0.15)
KPAD = 1280
SENT = T


def _stage_a(xl_ref, xm_ref, xr_ref, w0_ref, w1_ref, w2_ref, b_ref,
             ws_ref, bs_ref, tab_ref, sc_ref, cs_ref):
    i = pl.program_id(0)
    dn = (((0,), (1,)), ((), ()))
    acc = lax.dot_general(xl_ref[...], w0_ref[...], dn,
                          preferred_element_type=jnp.float32)
    acc += lax.dot_general(xm_ref[...], w1_ref[...], dn,
                           preferred_element_type=jnp.float32)
    acc += lax.dot_general(xr_ref[...], w2_ref[...], dn,
                           preferred_element_type=jnp.float32)
    h = jnp.maximum(acc + b_ref[...], 0.0)
    valid = jnp.where(i < NTILE, 1.0, 0.0)
    h = h * valid
    tab_ref[:, :C] = h
    tab_ref[:, C:] = jnp.transpose(xm_ref[...]) * valid
    s = jax.nn.sigmoid(jnp.dot(h, ws_ref[...],
                               preferred_element_type=jnp.float32)
                       + bs_ref[0, 0])
    sc_ref[0, 0, :] = s[:, 0] * valid
    cs_ref[0, 0, :] = jnp.sum(h, axis=1)


def _stage_b(sc_ref, cs_ref, picks_ref):
    s2 = sc_ref[...]
    cs = cs_ref[...]
    u = lax.bitcast_convert_type(s2, jnp.int32)

    def body(j, m):
        cand = m | (jnp.int32(1) << (jnp.int32(30) - j))
        cnt = jnp.sum(jnp.where(u >= cand, 1.0, 0.0))
        return jnp.where(cnt >= K, cand, m)

    m = lax.fori_loop(0, 31, body, jnp.int32(0))
    low = lax.bitcast_convert_type(m, jnp.float32)

    flag = jnp.logical_and(s2 >= low, cs != 0.0)
    F = flag.astype(jnp.float32)
    r = lax.broadcasted_iota(jnp.int32, (TILE, TILE), 0)
    c = lax.broadcasted_iota(jnp.int32, (TILE, TILE), 1)
    tri = (r <= c).astype(jnp.float32)
    cs1 = jnp.dot(F, tri, preferred_element_type=jnp.float32)
    rows = cs1[:, TILE - 1:TILE]
    r2 = lax.broadcasted_iota(jnp.int32, (NTILE + 1, NTILE + 1), 0)
    c2 = lax.broadcasted_iota(jnp.int32, (NTILE + 1, NTILE + 1), 1)
    tri2 = (c2 < r2).astype(jnp.float32)
    ex = jnp.dot(tri2, rows, preferred_element_type=jnp.float32)
    rank = cs1 + ex

    pv = 1.0 + lax.broadcasted_iota(jnp.int32, (KPAD, 1), 0).astype(jnp.float32)
    lane = lax.broadcasted_iota(jnp.int32, (1, TILE), 1).astype(jnp.float32)

    acc = jnp.zeros((KPAD,), jnp.float32)
    for j in range(NTILE + 1):
        crow = rank[j:j + 1, :]
        frow = F[j:j + 1, :]
        tglob = lane + jnp.float32(TILE * j)
        hit = jnp.logical_and(crow == pv, frow > 0.5)
        acc = acc + jnp.sum(jnp.where(hit, tglob, 0.0), axis=1)
    picks_ref[0, :] = acc.astype(jnp.int32)


def _stage_d(g_ref, a0_ref, a1_ref, a2_ref, b_ref, out_ref):
    gl = g_ref[0:KPAD, 0:C]
    gc = g_ref[KPAD:2 * KPAD, :]
    gr = g_ref[2 * KPAD:3 * KPAD, 0:C]
    acc = jnp.dot(gl, a0_ref[...], preferred_element_type=jnp.float32)
    acc += jnp.dot(gc[:, 0:C], a1_ref[...], preferred_element_type=jnp.float32)
    acc += jnp.dot(gr, a2_ref[...], preferred_element_type=jnp.float32)
    out_ref[...] = acc + b_ref[...] + gc[:, C:]


def _sc_gather(table, idx):
    info = plsc.get_sparse_core_info()
    nw = info.num_cores * info.num_subcores
    bpw = (3 * KPAD) // nw
    mesh = plsc.VectorSubcoreMesh(core_axis_name="c", subcore_axis_name="s")

    @functools.partial(
        pl.kernel, mesh=mesh,
        out_type=jax.ShapeDtypeStruct((3 * KPAD, 2 * C), jnp.float32),
        scratch_types=[
            pltpu.VMEM((bpw,), jnp.int32),
            pltpu.VMEM((bpw, 2 * C), jnp.float32),
            pltpu.SemaphoreType.DMA,
        ],
    )
    def k(table_hbm, idx_hbm, out_hbm, idx_v, rows_v, sem):
        wid = lax.axis_index("s") * info.num_cores + lax.axis_index("c")
        base = wid * bpw
        pltpu.sync_copy(idx_hbm.at[pl.ds(base, bpw)], idx_v)
        pltpu.async_copy(table_hbm.at[idx_v], rows_v, sem).wait()
        pltpu.sync_copy(rows_v, out_hbm.at[pl.ds(base, bpw)])

    return k(table, idx)


def kernel(x, W_fcsn, b_fcsn, W_score, b_score, W1, b1):
    x2 = x[0]
    zc = jnp.zeros((C, 1), jnp.float32)
    zpad = jnp.zeros((C, TILE - 1), jnp.float32)
    xm = jnp.concatenate([x2, jnp.zeros((C, TILE), jnp.float32)], axis=1)
    xl = jnp.concatenate([zc, x2, zpad], axis=1)
    xr = jnp.concatenate([x2[:, 1:], zc, zpad, zc], axis=1)

    w0, w1, w2 = W_fcsn[:, :, 0], W_fcsn[:, :, 1], W_fcsn[:, :, 2]
    ws = W_score[0, :, :]
    bs = b_score.reshape(1, 1)
    bf = b_fcsn.reshape(1, C)

    grid = NTILE + 1
    tab, sc17, cs17 = pl.pallas_call(
        _stage_a,
        grid=(grid,),
        in_specs=[
            pl.BlockSpec((C, TILE), lambda i: (0, i)),
            pl.BlockSpec((C, TILE), lambda i: (0, i)),
            pl.BlockSpec((C, TILE), lambda i: (0, i)),
            pl.BlockSpec((C, C), lambda i: (0, 0)),
            pl.BlockSpec((C, C), lambda i: (0, 0)),
            pl.BlockSpec((C, C), lambda i: (0, 0)),
            pl.BlockSpec((1, C), lambda i: (0, 0)),
            pl.BlockSpec((C, 1), lambda i: (0, 0)),
            pl.BlockSpec((1, 1), lambda i: (0, 0)),
        ],
        out_specs=[
            pl.BlockSpec((TILE, 2 * C), lambda i: (i, 0)),
            pl.BlockSpec((1, 1, TILE), lambda i: (i, 0, 0)),
            pl.BlockSpec((1, 1, TILE), lambda i: (i, 0, 0)),
        ],
        out_shape=[
            jax.ShapeDtypeStruct((TPAD, 2 * C), jnp.float32),
            jax.ShapeDtypeStruct((grid, 1, TILE), jnp.float32),
            jax.ShapeDtypeStruct((grid, 1, TILE), jnp.float32),
        ],
    )(xl, xm, xr, w0, w1, w2, bf, ws, bs)

    picks2 = pl.pallas_call(
        _stage_b,
        out_shape=jax.ShapeDtypeStruct((1, KPAD), jnp.int32),
    )(sc17.reshape(grid, TILE), cs17.reshape(grid, TILE))

    p0 = picks2[0]
    j = jnp.arange(KPAD, dtype=jnp.int32)
    pc = jnp.where(j < K, p0, SENT)
    sent1 = jnp.full((1,), SENT, jnp.int32)
    plft = jnp.concatenate([sent1, pc[:-1]])
    prgt = jnp.concatenate([pc[1:], sent1])
    idx_all = jnp.concatenate([plft, pc, prgt])

    g_all = _sc_gather(tab, idx_all)

    a0, a1, a2 = W1[:, :, 0].T, W1[:, :, 1].T, W1[:, :, 2].T
    outT = pl.pallas_call(
        _stage_d,
        out_shape=jax.ShapeDtypeStruct((KPAD, C), jnp.float32),
    )(g_all, a0, a1, a2, b1.reshape(1, C))

    s = jnp.transpose(outT[:K])[None]
    return (s, p0[:K])


# speedup vs baseline: 1.3574x; 1.2320x over previous
import functools

import jax
import jax.numpy as jnp
from jax import lax
from jax.experimental import pallas as pl
from jax.experimental.pallas import tpu as pltpu
from jax.experimental.pallas import tpu_sc as plsc

C = 256
T = 8192
TILE = 512
NTILE = T // TILE
TPAD = (NTILE + 1) * TILE
K = int(T * 0.15)
KPAD = 1280
SENT = T


def _stage_a(xp_ref, w0_ref, w1_ref, w2_ref, b_ref,
             ws_ref, bs_ref, tab_ref, sc_ref, cs_ref):
    i = pl.program_id(0)
    t0 = i * TILE
    dn = (((0,), (1,)), ((), ()))
    xw = xp_ref[:, pl.ds(t0, TILE + 128)]
    xm = lax.slice(xw, (0, 1), (C, TILE + 1))
    acc = lax.dot_general(lax.slice(xw, (0, 0), (C, TILE)), w0_ref[...], dn,
                          preferred_element_type=jnp.float32)
    acc += lax.dot_general(xm, w1_ref[...], dn,
                           preferred_element_type=jnp.float32)
    acc += lax.dot_general(lax.slice(xw, (0, 2), (C, TILE + 2)), w2_ref[...], dn,
                           preferred_element_type=jnp.float32)
    h = jnp.maximum(acc + b_ref[...], 0.0)
    valid = jnp.where(i < NTILE, 1.0, 0.0)
    h = h * valid
    tab_ref[:, :C] = h
    tab_ref[:, C:] = jnp.transpose(xm) * valid
    s = jax.nn.sigmoid(jnp.dot(h, ws_ref[...],
                               preferred_element_type=jnp.float32)
                       + bs_ref[0, 0])
    sc_ref[0, 0, :] = s[:, 0] * valid
    cs_ref[0, 0, :] = jnp.sum(h, axis=1)


def _stage_b(sc_ref, cs_ref, picks_ref):
    s2 = sc_ref[...]
    cs = cs_ref[...]
    u = lax.bitcast_convert_type(s2, jnp.int32)

    def body(j, m):
        cand = m | (jnp.int32(1) << (jnp.int32(30) - j))
        cnt = jnp.sum(jnp.where(u >= cand, 1.0, 0.0))
        return jnp.where(cnt >= K, cand, m)

    m = lax.fori_loop(0, 31, body, jnp.int32(0))
    low = lax.bitcast_convert_type(m, jnp.float32)

    flag = jnp.logical_and(s2 >= low, cs != 0.0)
    F = flag.astype(jnp.float32)
    r = lax.broadcasted_iota(jnp.int32, (TILE, TILE), 0)
    c = lax.broadcasted_iota(jnp.int32, (TILE, TILE), 1)
    tri = (r <= c).astype(jnp.float32)
    cs1 = jnp.dot(F, tri, preferred_element_type=jnp.float32)
    rows = cs1[:, TILE - 1:TILE]
    r2 = lax.broadcasted_iota(jnp.int32, (NTILE + 1, NTILE + 1), 0)
    c2 = lax.broadcasted_iota(jnp.int32, (NTILE + 1, NTILE + 1), 1)
    tri2 = (c2 < r2).astype(jnp.float32)
    ex = jnp.dot(tri2, rows, preferred_element_type=jnp.float32)
    rank = cs1 + ex

    pv = 1.0 + lax.broadcasted_iota(jnp.int32, (KPAD, 1), 0).astype(jnp.float32)
    lane = lax.broadcasted_iota(jnp.int32, (1, TILE), 1).astype(jnp.float32)

    acc = jnp.zeros((KPAD,), jnp.float32)
    for j in range(NTILE + 1):
        crow = rank[j:j + 1, :]
        frow = F[j:j + 1, :]
        tglob = lane + jnp.float32(TILE * j)
        hit = jnp.logical_and(crow == pv, frow > 0.5)
        acc = acc + jnp.sum(jnp.where(hit, tglob, 0.0), axis=1)
    picks_ref[0, :] = acc.astype(jnp.int32)


def _stage_d(g_ref, a0_ref, a1_ref, a2_ref, b_ref, out_ref):
    gl = g_ref[0:KPAD, 0:C]
    gc = g_ref[1:KPAD + 1, :]
    gr = g_ref[2:KPAD + 2, 0:C]
    acc = jnp.dot(gl, a0_ref[...], preferred_element_type=jnp.float32)
    acc += jnp.dot(gc[:, 0:C], a1_ref[...], preferred_element_type=jnp.float32)
    acc += jnp.dot(gr, a2_ref[...], preferred_element_type=jnp.float32)
    out_ref[...] = acc + b_ref[...] + gc[:, C:]


GPAD = 1536


def _sc_gather(table, idx):
    info = plsc.get_sparse_core_info()
    nw = info.num_cores * info.num_subcores
    bpw = GPAD // nw
    mesh = plsc.VectorSubcoreMesh(core_axis_name="c", subcore_axis_name="s")

    @functools.partial(
        pl.kernel, mesh=mesh,
        out_type=jax.ShapeDtypeStruct((GPAD, 2 * C), jnp.float32),
        scratch_types=[
            pltpu.VMEM((bpw,), jnp.int32),
            pltpu.VMEM((bpw, 2 * C), jnp.float32),
            pltpu.SemaphoreType.DMA,
        ],
    )
    def k(table_hbm, idx_hbm, out_hbm, idx_v, rows_v, sem):
        wid = lax.axis_index("s") * info.num_cores + lax.axis_index("c")
        base = wid * bpw
        pltpu.sync_copy(idx_hbm.at[pl.ds(base, bpw)], idx_v)
        pltpu.async_copy(table_hbm.at[idx_v], rows_v, sem).wait()
        pltpu.sync_copy(rows_v, out_hbm.at[pl.ds(base, bpw)])

    return k(table, idx)


def kernel(x, W_fcsn, b_fcsn, W_score, b_score, W1, b1):
    x2 = x[0]
    xp = jnp.pad(x2, ((0, 0), (1, TPAD + 128 - 1 - T)))

    w0, w1, w2 = W_fcsn[:, :, 0], W_fcsn[:, :, 1], W_fcsn[:, :, 2]
    ws = W_score[0, :, :]
    bs = b_score.reshape(1, 1)
    bf = b_fcsn.reshape(1, C)

    grid = NTILE + 1
    tab, sc17, cs17 = pl.pallas_call(
        _stage_a,
        grid=(grid,),
        in_specs=[
            pl.BlockSpec((C, TPAD + 128), lambda i: (0, 0)),
            pl.BlockSpec((C, C), lambda i: (0, 0)),
            pl.BlockSpec((C, C), lambda i: (0, 0)),
            pl.BlockSpec((C, C), lambda i: (0, 0)),
            pl.BlockSpec((1, C), lambda i: (0, 0)),
            pl.BlockSpec((C, 1), lambda i: (0, 0)),
            pl.BlockSpec((1, 1), lambda i: (0, 0)),
        ],
        out_specs=[
            pl.BlockSpec((TILE, 2 * C), lambda i: (i, 0)),
            pl.BlockSpec((1, 1, TILE), lambda i: (i, 0, 0)),
            pl.BlockSpec((1, 1, TILE), lambda i: (i, 0, 0)),
        ],
        out_shape=[
            jax.ShapeDtypeStruct((TPAD, 2 * C), jnp.float32),
            jax.ShapeDtypeStruct((grid, 1, TILE), jnp.float32),
            jax.ShapeDtypeStruct((grid, 1, TILE), jnp.float32),
        ],
    )(xp, w0, w1, w2, bf, ws, bs)

    picks2 = pl.pallas_call(
        _stage_b,
        out_shape=jax.ShapeDtypeStruct((1, KPAD), jnp.int32),
    )(sc17.reshape(grid, TILE), cs17.reshape(grid, TILE))

    p0 = picks2[0]
    j = jnp.arange(KPAD, dtype=jnp.int32)
    pc = jnp.where(j < K, p0, SENT)
    sent1 = jnp.full((1,), SENT, jnp.int32)
    sent_pad = jnp.full((GPAD - KPAD - 1,), SENT, jnp.int32)
    idx_all = jnp.concatenate([sent1, pc, sent_pad])

    g_all = _sc_gather(tab, idx_all)

    a0, a1, a2 = W1[:, :, 0].T, W1[:, :, 1].T, W1[:, :, 2].T
    outT = pl.pallas_call(
        _stage_d,
        out_shape=jax.ShapeDtypeStruct((KPAD, C), jnp.float32),
    )(g_all, a0, a1, a2, b1.reshape(1, C))

    s = jnp.transpose(outT[:K])[None]
    return (s, p0[:K])


# speedup vs baseline: 1.4115x; 1.0398x over previous
import functools

import jax
import jax.numpy as jnp
from jax import lax
from jax.experimental import pallas as pl
from jax.experimental.pallas import tpu as pltpu
from jax.experimental.pallas import tpu_sc as plsc

C = 256
T = 8192
TILE = 1024
NTILE = T // TILE
TPAD = (NTILE + 1) * TILE
K = int(T * 0.15)
KPAD = 1280
SENT = T
GPAD = 1536


def _stage_a(xp_ref, w0_ref, w1_ref, w2_ref, b_ref,
             ws_ref, bs_ref, tab_ref, sc_ref, cs_ref):
    i = pl.program_id(0)
    t0 = i * TILE
    dn = (((0,), (1,)), ((), ()))
    xw = xp_ref[:, pl.ds(t0, TILE + 128)]
    xm = lax.slice(xw, (0, 1), (C, TILE + 1))
    acc = lax.dot_general(lax.slice(xw, (0, 0), (C, TILE)), w0_ref[...], dn,
                          preferred_element_type=jnp.float32)
    acc += lax.dot_general(xm, w1_ref[...], dn,
                           preferred_element_type=jnp.float32)
    acc += lax.dot_general(lax.slice(xw, (0, 2), (C, TILE + 2)), w2_ref[...], dn,
                           preferred_element_type=jnp.float32)
    h = jnp.maximum(acc + b_ref[...], 0.0)
    valid = jnp.where(i < NTILE, 1.0, 0.0)
    h = h * valid
    tab_ref[:, :C] = h
    r = lax.broadcasted_iota(jnp.int32, (C, C), 0)
    c = lax.broadcasted_iota(jnp.int32, (C, C), 1)
    eye = jnp.where(r == c, valid, 0.0)
    dn2 = (((0,), (0,)), ((), ()))
    tab_ref[:, C:] = lax.dot_general(xm, eye, dn2,
                                     preferred_element_type=jnp.float32)
    s = jax.nn.sigmoid(jnp.dot(h, ws_ref[...],
                               preferred_element_type=jnp.float32)
                       + bs_ref[0, 0])
    sc_ref[0, 0, :] = s[:, 0] * valid
    cs_ref[0, 0, :] = jnp.sum(h, axis=1)


def _stage_b(sc_ref, cs_ref, idx_ref):
    s2 = sc_ref[...]
    cs = cs_ref[...]
    u = lax.bitcast_convert_type(s2, jnp.int32)

    def body(j, m):
        cand = m | (jnp.int32(1) << (jnp.int32(30) - j))
        cnt = jnp.sum(jnp.where(u >= cand, 1.0, 0.0))
        return jnp.where(cnt >= K, cand, m)

    m = lax.fori_loop(0, 31, body, jnp.int32(0))
    low = lax.bitcast_convert_type(m, jnp.float32)

    flag = jnp.logical_and(s2 >= low, cs != 0.0)
    F = flag.astype(jnp.float32)
    r = lax.broadcasted_iota(jnp.int32, (TILE, TILE), 0)
    c = lax.broadcasted_iota(jnp.int32, (TILE, TILE), 1)
    tri = (r <= c).astype(jnp.float32)
    cs1 = jnp.dot(F, tri, preferred_element_type=jnp.float32)
    rows = cs1[:, TILE - 1:TILE]
    n = NTILE + 1
    r2 = lax.broadcasted_iota(jnp.int32, (n, n), 0)
    c2 = lax.broadcasted_iota(jnp.int32, (n, n), 1)
    tri2 = (c2 < r2).astype(jnp.float32)
    ex = jnp.dot(tri2, rows, preferred_element_type=jnp.float32)
    rank = cs1 + ex

    jv = lax.broadcasted_iota(jnp.int32, (GPAD, 1), 0)
    pv = jv.astype(jnp.float32)
    lane = lax.broadcasted_iota(jnp.int32, (1, TILE), 1).astype(jnp.float32)

    acc = jnp.zeros((GPAD,), jnp.float32)
    for j in range(n):
        crow = rank[j:j + 1, :]
        frow = F[j:j + 1, :]
        tglob = lane + jnp.float32(TILE * j)
        hit = jnp.logical_and(crow == pv, frow > 0.5)
        acc = acc + jnp.sum(jnp.where(hit, tglob, 0.0), axis=1)

    in_range = jnp.logical_and(jv[:, 0] >= 1, jv[:, 0] <= K)
    idx_ref[0, :] = jnp.where(in_range, acc.astype(jnp.int32), SENT)


def _stage_d(g_ref, a0_ref, a1_ref, a2_ref, b_ref, out_ref):
    gl = g_ref[0:KPAD, 0:C]
    gc = g_ref[1:KPAD + 1, :]
    gr = g_ref[2:KPAD + 2, 0:C]
    acc = jnp.dot(gl, a0_ref[...], preferred_element_type=jnp.float32)
    acc += jnp.dot(gc[:, 0:C], a1_ref[...], preferred_element_type=jnp.float32)
    acc += jnp.dot(gr, a2_ref[...], preferred_element_type=jnp.float32)
    out_ref[...] = acc + b_ref[...] + gc[:, C:]


def _sc_gather(table, idx):
    info = plsc.get_sparse_core_info()
    nw = info.num_cores * info.num_subcores
    bpw = GPAD // nw
    mesh = plsc.VectorSubcoreMesh(core_axis_name="c", subcore_axis_name="s")

    @functools.partial(
        pl.kernel, mesh=mesh,
        out_type=jax.ShapeDtypeStruct((GPAD, 2 * C), jnp.float32),
        scratch_types=[
            pltpu.VMEM((bpw,), jnp.int32),
            pltpu.VMEM((bpw, 2 * C), jnp.float32),
            pltpu.SemaphoreType.DMA,
        ],
    )
    def k(table_hbm, idx_hbm, out_hbm, idx_v, rows_v, sem):
        wid = lax.axis_index("s") * info.num_cores + lax.axis_index("c")
        base = wid * bpw
        pltpu.sync_copy(idx_hbm.at[pl.ds(base, bpw)], idx_v)
        pltpu.async_copy(table_hbm.at[idx_v], rows_v, sem).wait()
        pltpu.sync_copy(rows_v, out_hbm.at[pl.ds(base, bpw)])

    return k(table, idx)


def kernel(x, W_fcsn, b_fcsn, W_score, b_score, W1, b1):
    x2 = x[0]
    xp = jnp.pad(x2, ((0, 0), (1, TPAD + 128 - 1 - T)))

    w0, w1, w2 = W_fcsn[:, :, 0], W_fcsn[:, :, 1], W_fcsn[:, :, 2]
    ws = W_score[0, :, :]
    bs = b_score.reshape(1, 1)
    bf = b_fcsn.reshape(1, C)

    grid = NTILE + 1
    tab, sc9, cs9 = pl.pallas_call(
        _stage_a,
        grid=(grid,),
        in_specs=[
            pl.BlockSpec((C, TPAD + 128), lambda i: (0, 0)),
            pl.BlockSpec((C, C), lambda i: (0, 0)),
            pl.BlockSpec((C, C), lambda i: (0, 0)),
            pl.BlockSpec((C, C), lambda i: (0, 0)),
            pl.BlockSpec((1, C), lambda i: (0, 0)),
            pl.BlockSpec((C, 1), lambda i: (0, 0)),
            pl.BlockSpec((1, 1), lambda i: (0, 0)),
        ],
        out_specs=[
            pl.BlockSpec((TILE, 2 * C), lambda i: (i, 0)),
            pl.BlockSpec((1, 1, TILE), lambda i: (i, 0, 0)),
            pl.BlockSpec((1, 1, TILE), lambda i: (i, 0, 0)),
        ],
        out_shape=[
            jax.ShapeDtypeStruct((TPAD, 2 * C), jnp.float32),
            jax.ShapeDtypeStruct((grid, 1, TILE), jnp.float32),
            jax.ShapeDtypeStruct((grid, 1, TILE), jnp.float32),
        ],
    )(xp, w0, w1, w2, bf, ws, bs)

    idx2 = pl.pallas_call(
        _stage_b,
        out_shape=jax.ShapeDtypeStruct((1, GPAD), jnp.int32),
    )(sc9.reshape(grid, TILE), cs9.reshape(grid, TILE))

    g_all = _sc_gather(tab, idx2[0])

    a0, a1, a2 = W1[:, :, 0].T, W1[:, :, 1].T, W1[:, :, 2].T
    outT = pl.pallas_call(
        _stage_d,
        out_shape=jax.ShapeDtypeStruct((KPAD, C), jnp.float32),
    )(g_all, a0, a1, a2, b1.reshape(1, C))

    s = jnp.transpose(outT[:K])[None]
    return (s, idx2[0, 1:K + 1])


# speedup vs baseline: 1.4353x; 1.0169x over previous
import functools

import jax
import jax.numpy as jnp
from jax import lax
from jax.experimental import pallas as pl
from jax.experimental.pallas import tpu as pltpu
from jax.experimental.pallas import tpu_sc as plsc

C = 256
T = 8192
TILE = 1024
NTILE = T // TILE
TPAD = (NTILE + 1) * TILE
K = int(T * 0.15)
KPAD = 1280
SENT = T
GPAD = 1536


def _stage_a(xp_ref, w_ref, b_ref, ws_ref, bs_ref, tab_ref, sc_ref, cs_ref):
    i = pl.program_id(0)
    t0 = i * TILE
    dn = (((0,), (1,)), ((), ()))
    xw = xp_ref[:, pl.ds(t0, TILE + 128)]
    xm = lax.slice(xw, (0, 1), (C, TILE + 1))
    acc = lax.dot_general(lax.slice(xw, (0, 0), (C, TILE)), w_ref[0], dn,
                          preferred_element_type=jnp.float32)
    acc += lax.dot_general(xm, w_ref[1], dn,
                           preferred_element_type=jnp.float32)
    acc += lax.dot_general(lax.slice(xw, (0, 2), (C, TILE + 2)), w_ref[2], dn,
                           preferred_element_type=jnp.float32)
    h = jnp.maximum(acc + b_ref[...], 0.0)
    valid = jnp.where(i < NTILE, 1.0, 0.0)
    h = h * valid
    tab_ref[:, :C] = h
    r = lax.broadcasted_iota(jnp.int32, (C, C), 0)
    c = lax.broadcasted_iota(jnp.int32, (C, C), 1)
    eye = jnp.where(r == c, valid, 0.0)
    dn2 = (((0,), (0,)), ((), ()))
    tab_ref[:, C:] = lax.dot_general(xm, eye, dn2,
                                     preferred_element_type=jnp.float32)
    s = jax.nn.sigmoid(jnp.dot(h, ws_ref[...],
                               preferred_element_type=jnp.float32)
                       + bs_ref[0, 0])
    sc_ref[0, 0, :] = s[:, 0] * valid
    cs_ref[0, 0, :] = jnp.sum(h, axis=1)


def _stage_b(sc_ref, cs_ref, idx_ref):
    s2 = sc_ref[...]
    cs = cs_ref[...]
    u = lax.bitcast_convert_type(s2, jnp.int32)

    def body(j, m):
        cand = m | (jnp.int32(1) << (jnp.int32(30) - j))
        cnt = jnp.sum(jnp.where(u >= cand, 1.0, 0.0))
        return jnp.where(cnt >= K, cand, m)

    m = lax.fori_loop(0, 31, body, jnp.int32(0))
    low = lax.bitcast_convert_type(m, jnp.float32)

    flag = jnp.logical_and(s2 >= low, cs != 0.0)
    F = flag.astype(jnp.float32)
    r = lax.broadcasted_iota(jnp.int32, (TILE, TILE), 0)
    c = lax.broadcasted_iota(jnp.int32, (TILE, TILE), 1)
    tri = (r <= c).astype(jnp.float32)
    cs1 = jnp.dot(F, tri, preferred_element_type=jnp.float32)
    rows = cs1[:, TILE - 1:TILE]
    n = NTILE + 1
    r2 = lax.broadcasted_iota(jnp.int32, (n, n), 0)
    c2 = lax.broadcasted_iota(jnp.int32, (n, n), 1)
    tri2 = (c2 < r2).astype(jnp.float32)
    ex = jnp.dot(tri2, rows, preferred_element_type=jnp.float32)
    rank = cs1 + ex

    jv = lax.broadcasted_iota(jnp.int32, (GPAD, 1), 0)
    pv = jv.astype(jnp.float32)
    lane = lax.broadcasted_iota(jnp.int32, (1, TILE), 1).astype(jnp.float32)

    acc = jnp.zeros((GPAD,), jnp.float32)
    for j in range(n):
        crow = rank[j:j + 1, :]
        frow = F[j:j + 1, :]
        tglob = lane + jnp.float32(TILE * j)
        hit = jnp.logical_and(crow == pv, frow > 0.5)
        acc = acc + jnp.sum(jnp.where(hit, tglob, 0.0), axis=1)

    in_range = jnp.logical_and(jv[:, 0] >= 1, jv[:, 0] <= K)
    idx_ref[0, :] = jnp.where(in_range, acc.astype(jnp.int32), SENT)


def _stage_d(g_ref, w_ref, b_ref, out_ref):
    gl = g_ref[0:KPAD, 0:C]
    gc = g_ref[1:KPAD + 1, :]
    gr = g_ref[2:KPAD + 2, 0:C]
    dn = (((1,), (1,)), ((), ()))
    acc = lax.dot_general(gl, w_ref[0], dn, preferred_element_type=jnp.float32)
    acc += lax.dot_general(gc[:, 0:C], w_ref[1], dn,
                           preferred_element_type=jnp.float32)
    acc += lax.dot_general(gr, w_ref[2], dn, preferred_element_type=jnp.float32)
    out_ref[...] = acc + b_ref[...] + gc[:, C:]


def _sc_gather(table, idx):
    info = plsc.get_sparse_core_info()
    nw = info.num_cores * info.num_subcores
    bpw = GPAD // nw
    mesh = plsc.VectorSubcoreMesh(core_axis_name="c", subcore_axis_name="s")

    @functools.partial(
        pl.kernel, mesh=mesh,
        out_type=jax.ShapeDtypeStruct((GPAD, 2 * C), jnp.float32),
        scratch_types=[
            pltpu.VMEM((bpw,), jnp.int32),
            pltpu.VMEM((bpw, 2 * C), jnp.float32),
            pltpu.SemaphoreType.DMA,
        ],
    )
    def k(table_hbm, idx_hbm, out_hbm, idx_v, rows_v, sem):
        wid = lax.axis_index("s") * info.num_cores + lax.axis_index("c")
        base = wid * bpw
        pltpu.sync_copy(idx_hbm.at[pl.ds(base, bpw)], idx_v)
        pltpu.async_copy(table_hbm.at[idx_v], rows_v, sem).wait()
        pltpu.sync_copy(rows_v, out_hbm.at[pl.ds(base, bpw)])

    return k(table, idx)


def kernel(x, W_fcsn, b_fcsn, W_score, b_score, W1, b1):
    x2 = x[0]
    xp = jnp.pad(x2, ((0, 0), (1, TPAD + 128 - 1 - T)))

    wf = W_fcsn.transpose(2, 0, 1)
    ws = W_score[0, :, :]
    bs = b_score.reshape(1, 1)
    bf = b_fcsn.reshape(1, C)

    grid = NTILE + 1
    tab, sc9, cs9 = pl.pallas_call(
        _stage_a,
        grid=(grid,),
        in_specs=[
            pl.BlockSpec((C, TPAD + 128), lambda i: (0, 0)),
            pl.BlockSpec((3, C, C), lambda i: (0, 0, 0)),
            pl.BlockSpec((1, C), lambda i: (0, 0)),
            pl.BlockSpec((C, 1), lambda i: (0, 0)),
            pl.BlockSpec((1, 1), lambda i: (0, 0)),
        ],
        out_specs=[
            pl.BlockSpec((TILE, 2 * C), lambda i: (i, 0)),
            pl.BlockSpec((1, 1, TILE), lambda i: (i, 0, 0)),
            pl.BlockSpec((1, 1, TILE), lambda i: (i, 0, 0)),
        ],
        out_shape=[
            jax.ShapeDtypeStruct((TPAD, 2 * C), jnp.float32),
            jax.ShapeDtypeStruct((grid, 1, TILE), jnp.float32),
            jax.ShapeDtypeStruct((grid, 1, TILE), jnp.float32),
        ],
    )(xp, wf, bf, ws, bs)

    idx1 = pl.pallas_call(
        _stage_b,
        out_shape=jax.ShapeDtypeStruct((1, GPAD), jnp.int32),
    )(sc9.reshape(grid, TILE), cs9.reshape(grid, TILE))[0]

    g_all = _sc_gather(tab, idx1)

    outT = pl.pallas_call(
        _stage_d,
        out_shape=jax.ShapeDtypeStruct((KPAD, C), jnp.float32),
    )(g_all, W1.transpose(2, 0, 1), b1.reshape(1, C))

    s = jnp.transpose(outT[:K])[None]
    return (s, idx1[1:K + 1])
